# baseline (speedup 1.0000x reference)
import jax
import jax.numpy as jnp
from jax import lax
from jax.experimental import pallas as pl
from jax.experimental.pallas import tpu as pltpu
from jax.experimental.pallas import tpu_sc as plsc

_M = 16384
_D = 4096
_B = 4096

_INFO = plsc.get_sparse_core_info()
_NC = _INFO.num_cores
_NS = _INFO.num_subcores
_NW = _NC * _NS
_L = 16

_ROWS_PER_TILE = _M // _NW
_R = 8
_LIST_ROWS = _ROWS_PER_TILE // _R + 2


def _body(mem_hbm, idx_hbm, val_hbm, out_hbm,
          idx_v, p_ref, blist, mlist, ulist, buf0, buf1, buf2,
          sem_g0, sem_g1, sem_g2, sem_s0, sem_s1, sem_s2):
    wid = lax.axis_index("s") * _NC + lax.axis_index("c")
    r0 = wid * _ROWS_PER_TILE
    lane = lax.iota(jnp.int32, _L)

    pltpu.sync_copy(idx_hbm, idx_v)

    def init_body(i, carry):
        plsc.store_scatter(p_ref, [i * _L + lane], jnp.full((_L,), -1, jnp.int32))
        return carry
    lax.fori_loop(0, _ROWS_PER_TILE // _L, init_body, 0)

    lane_next = jnp.minimum(lane + 1, _L - 1)


    def scan_body(g, carry):
        idx_g = plsc.load_gather(idx_v, [g * _L + lane])
        ks = jnp.sort(idx_g * _L + lane)
        nxt = ks[lane_next]
        ms = ks >> 4
        keep = (ms != (nxt >> 4)) | (lane == _L - 1)
        mloc = ms - r0
        keep = keep & (mloc >= 0) & (mloc < _ROWS_PER_TILE)
        bs = g * _L + (ks & (_L - 1))
        plsc.store_scatter(p_ref, [mloc & (_ROWS_PER_TILE - 1)], bs, mask=keep)
        return carry
    lax.fori_loop(0, _B // _L, scan_body, 0, unroll=4)

    def compress_body(g, carry):
        cw, cu, comb, um = carry
        pv = plsc.load_gather(p_ref, [g * _L + lane])
        m_vec = r0 + g * _L + lane
        wr = pv >= 0
        wr_i = wr.astype(jnp.int32)
        uw_i = 1 - wr_i
        incl_w = plsc.cumsum(wr_i)
        incl_u = plsc.cumsum(uw_i)
        pos_w = jnp.full((_L,), cw, jnp.int32) + incl_w - wr_i
        pos_u = jnp.full((_L,), cu, jnp.int32) + incl_u - uw_i
        plsc.store_scatter(blist, [pos_w // _R, pos_w % _R], pv, mask=wr)
        plsc.store_scatter(mlist, [pos_w // _R, pos_w % _R], m_vec, mask=wr)
        plsc.store_scatter(ulist, [pos_u // _R, pos_u % _R], m_vec,
                           mask=jnp.logical_not(wr))
        comb = jnp.maximum(comb, jnp.max(jnp.where(wr, (m_vec << 12) | pv, -1)))
        um = jnp.maximum(um, jnp.max(jnp.where(wr, -1, m_vec)))
        return cw + jnp.max(incl_w), cu + jnp.max(incl_u), comb, um
    cw, cu, comb, um = lax.fori_loop(
        0, _ROWS_PER_TILE // _L, compress_body,
        (jnp.int32(0), jnp.int32(0), jnp.int32(-1), jnp.int32(-1)))

    def _pad(list_refs, vals, cnt):
        npad = (_R - cnt % _R) % _R
        posv = jnp.full((_L,), cnt, jnp.int32) + lane
        padmask = lane < npad
        for ref, v in zip(list_refs, vals):
            plsc.store_scatter(ref, [posv // _R, posv % _R],
                               jnp.full((_L,), v, jnp.int32), mask=padmask)
        return (cnt + npad) // _R

    def _pipe(src_hbm, slist, dlist, ng):
        bufs = (buf0, buf1, buf2)
        gsems = (sem_g0, sem_g1, sem_g2)
        ssems = (sem_s0, sem_s1, sem_s2)

        def _step(g, k):
            @pl.when(g >= 3)
            def _():
                pltpu.make_async_copy(bufs[k], out_hbm.at[dlist.at[g]],
                                      ssems[k]).wait()
            pltpu.make_async_copy(src_hbm.at[slist.at[g]], bufs[k],
                                  gsems[k]).start()
            kp = (k + 2) % 3
            @pl.when(g >= 1)
            def _():
                pltpu.make_async_copy(src_hbm.at[slist.at[g]], bufs[kp],
                                      gsems[kp]).wait()
                pltpu.make_async_copy(bufs[kp], out_hbm.at[dlist.at[g - 1]],
                                      ssems[kp]).start()

        def body(g3, carry):
            for k in range(3):
                g = 3 * g3 + k
                @pl.when(g < ng)
                def _(g=g, k=k):
                    _step(g, k)
            return carry
        lax.fori_loop(0, (ng + 2) // 3, body, 0)

        for k in range(3):
            @pl.when((ng - 1) % 3 == k)
            def _(k=k):
                pltpu.make_async_copy(src_hbm.at[slist.at[0]], bufs[k],
                                      gsems[k]).wait()
                pltpu.make_async_copy(bufs[k], out_hbm.at[dlist.at[ng - 1]],
                                      ssems[k]).start()
        for k in range(3):
            @pl.when(k < ng)
            def _(k=k):
                pltpu.make_async_copy(bufs[k], out_hbm.at[dlist.at[0]],
                                      ssems[k]).wait()

    @pl.when(cu > 0)
    def _untouched_phase():
        ngu = _pad([ulist], [um], cu)
        _pipe(mem_hbm, ulist, ulist, ngu)

    @pl.when(cw > 0)
    def _written_phase():
        ngw = _pad([blist, mlist], [comb & 4095, comb >> 12], cw)
        _pipe(val_hbm, blist, mlist, ngw)


def kernel(mem, idx, val):
    mesh = plsc.VectorSubcoreMesh(core_axis_name="c", subcore_axis_name="s")
    f = pl.kernel(
        _body,
        out_type=jax.ShapeDtypeStruct((_M, _D), jnp.float32),
        mesh=mesh,
        compiler_params=pltpu.CompilerParams(needs_layout_passes=False),
        scratch_types=[
            pltpu.VMEM((_B,), jnp.int32),
            pltpu.VMEM((_ROWS_PER_TILE,), jnp.int32),
            pltpu.VMEM((_LIST_ROWS, _R), jnp.int32),
            pltpu.VMEM((_LIST_ROWS, _R), jnp.int32),
            pltpu.VMEM((_LIST_ROWS, _R), jnp.int32),
            pltpu.VMEM((_R, _D), jnp.float32),
            pltpu.VMEM((_R, _D), jnp.float32),
            pltpu.VMEM((_R, _D), jnp.float32),
            pltpu.SemaphoreType.DMA,
            pltpu.SemaphoreType.DMA,
            pltpu.SemaphoreType.DMA,
            pltpu.SemaphoreType.DMA,
            pltpu.SemaphoreType.DMA,
            pltpu.SemaphoreType.DMA,
        ],
    )
    return f(mem, idx.astype(jnp.int32), val)


# speedup vs baseline: 1.0039x; 1.0039x over previous
import jax
import jax.numpy as jnp
from jax import lax
from jax.experimental import pallas as pl
from jax.experimental.pallas import tpu as pltpu
from jax.experimental.pallas import tpu_sc as plsc

_M = 16384
_D = 4096
_B = 4096

_INFO = plsc.get_sparse_core_info()
_NC = _INFO.num_cores
_NS = _INFO.num_subcores
_NW = _NC * _NS
_L = 16

_ROWS_PER_TILE = _M // _NW
_R = 8
_LIST_ROWS = _ROWS_PER_TILE // _R + 2


def _body(mem_hbm, idx_hbm, val_hbm, out_hbm,
          idx_v, p_ref, blist, mlist, ulist, buf0, buf1, buf2,
          sem_g0, sem_g1, sem_g2, sem_s0, sem_s1, sem_s2):
    wid = lax.axis_index("s") * _NC + lax.axis_index("c")
    r0 = wid * _ROWS_PER_TILE
    lane = lax.iota(jnp.int32, _L)

    pltpu.sync_copy(idx_hbm, idx_v)

    def init_body(i, carry):
        plsc.store_scatter(p_ref, [i * _L + lane], jnp.full((_L,), -1, jnp.int32))
        return carry
    lax.fori_loop(0, _ROWS_PER_TILE // _L, init_body, 0)

    lane_next = jnp.minimum(lane + 1, _L - 1)


    def scan_body(g, carry):
        idx_g = plsc.load_gather(idx_v, [g * _L + lane])
        ks = jnp.sort(idx_g * _L + lane)
        nxt = ks[lane_next]
        ms = ks >> 4
        keep = (ms != (nxt >> 4)) | (lane == _L - 1)
        mloc = ms - r0
        keep = keep & (mloc >= 0) & (mloc < _ROWS_PER_TILE)
        bs = g * _L + (ks & (_L - 1))
        plsc.store_scatter(p_ref, [mloc & (_ROWS_PER_TILE - 1)], bs, mask=keep)
        return carry
    lax.fori_loop(0, _B // _L, scan_body, 0, unroll=4)

    def compress_body(g, carry):
        cw, cu, comb, um = carry
        pv = plsc.load_gather(p_ref, [g * _L + lane])
        m_vec = r0 + g * _L + lane
        wr = pv >= 0
        wr_i = wr.astype(jnp.int32)
        uw_i = 1 - wr_i
        incl_w = plsc.cumsum(wr_i)
        incl_u = plsc.cumsum(uw_i)
        pos_w = jnp.full((_L,), cw, jnp.int32) + incl_w - wr_i
        pos_u = jnp.full((_L,), cu, jnp.int32) + incl_u - uw_i
        plsc.store_scatter(blist, [pos_w // _R, pos_w % _R], pv, mask=wr)
        plsc.store_scatter(mlist, [pos_w // _R, pos_w % _R], m_vec, mask=wr)
        plsc.store_scatter(ulist, [pos_u // _R, pos_u % _R], m_vec,
                           mask=jnp.logical_not(wr))
        comb = jnp.maximum(comb, jnp.max(jnp.where(wr, (m_vec << 12) | pv, -1)))
        um = jnp.maximum(um, jnp.max(jnp.where(wr, -1, m_vec)))
        return cw + jnp.max(incl_w), cu + jnp.max(incl_u), comb, um
    cw, cu, comb, um = lax.fori_loop(
        0, _ROWS_PER_TILE // _L, compress_body,
        (jnp.int32(0), jnp.int32(0), jnp.int32(-1), jnp.int32(-1)))

    def _pad(list_refs, vals, cnt):
        npad = (_R - cnt % _R) % _R
        posv = jnp.full((_L,), cnt, jnp.int32) + lane
        padmask = lane < npad
        for ref, v in zip(list_refs, vals):
            plsc.store_scatter(ref, [posv // _R, posv % _R],
                               jnp.full((_L,), v, jnp.int32), mask=padmask)
        return (cnt + npad) // _R

    def _pipe(ngu, ng):
        bufs = (buf0, buf1, buf2)
        gsems = (sem_g0, sem_g1, sem_g2)
        ssems = (sem_s0, sem_s1, sem_s2)

        def _gather_start(g, k):
            @pl.when(g < ngu)
            def _():
                pltpu.make_async_copy(mem_hbm.at[ulist.at[g]], bufs[k],
                                      gsems[k]).start()
            @pl.when(g >= ngu)
            def _():
                pltpu.make_async_copy(val_hbm.at[blist.at[g - ngu]], bufs[k],
                                      gsems[k]).start()

        def _gather_wait(k):
            pltpu.make_async_copy(mem_hbm.at[ulist.at[0]], bufs[k],
                                  gsems[k]).wait()

        def _scatter_start(g, k):
            @pl.when(g < ngu)
            def _():
                pltpu.make_async_copy(bufs[k], out_hbm.at[ulist.at[g]],
                                      ssems[k]).start()
            @pl.when(g >= ngu)
            def _():
                pltpu.make_async_copy(bufs[k], out_hbm.at[mlist.at[g - ngu]],
                                      ssems[k]).start()

        def _scatter_wait(k):
            pltpu.make_async_copy(bufs[k], out_hbm.at[ulist.at[0]],
                                  ssems[k]).wait()

        def _step(g, k):
            @pl.when(g >= 3)
            def _():
                _scatter_wait(k)
            _gather_start(g, k)
            kp = (k + 2) % 3
            @pl.when(g >= 1)
            def _():
                _gather_wait(kp)
                _scatter_start(g - 1, kp)

        def body(g3, carry):
            for k in range(3):
                g = 3 * g3 + k
                @pl.when(g < ng)
                def _(g=g, k=k):
                    _step(g, k)
            return carry
        lax.fori_loop(0, (ng + 2) // 3, body, 0)

        for k in range(3):
            @pl.when((ng - 1) % 3 == k)
            def _(k=k):
                _gather_wait(k)
                _scatter_start(ng - 1, k)
        for k in range(3):
            @pl.when(k < ng)
            def _(k=k):
                _scatter_wait(k)

    ngu = _pad([ulist], [um], cu)
    ngw = _pad([blist, mlist], [comb & 4095, comb >> 12], cw)
    _pipe(ngu, ngu + ngw)


def kernel(mem, idx, val):
    mesh = plsc.VectorSubcoreMesh(core_axis_name="c", subcore_axis_name="s")
    f = pl.kernel(
        _body,
        out_type=jax.ShapeDtypeStruct((_M, _D), jnp.float32),
        mesh=mesh,
        compiler_params=pltpu.CompilerParams(needs_layout_passes=False),
        scratch_types=[
            pltpu.VMEM((_B,), jnp.int32),
            pltpu.VMEM((_ROWS_PER_TILE,), jnp.int32),
            pltpu.VMEM((_LIST_ROWS, _R), jnp.int32),
            pltpu.VMEM((_LIST_ROWS, _R), jnp.int32),
            pltpu.VMEM((_LIST_ROWS, _R), jnp.int32),
            pltpu.VMEM((_R, _D), jnp.float32),
            pltpu.VMEM((_R, _D), jnp.float32),
            pltpu.VMEM((_R, _D), jnp.float32),
            pltpu.SemaphoreType.DMA,
            pltpu.SemaphoreType.DMA,
            pltpu.SemaphoreType.DMA,
            pltpu.SemaphoreType.DMA,
            pltpu.SemaphoreType.DMA,
            pltpu.SemaphoreType.DMA,
        ],
    )
    return f(mem, idx.astype(jnp.int32), val)
